# Initial kernel scaffold; baseline (speedup 1.0000x reference)
import jax
import jax.numpy as jnp
from jax.experimental import pallas as pl


def kernel(x, G2_edge_attr, G1_edge_attr_matrix, G3_edge_index, G3_edge_attr, Wl1, Wr1, We1, att1, b1, Wl2, Wr2, We2, att2, b2):
    raise NotImplementedError("write your pallas kernel here")


# speedup vs baseline: 6.7897x; 6.7897x over previous
import functools

import jax
import jax.numpy as jnp
from jax import lax
from jax.experimental import pallas as pl
from jax.experimental.pallas import tpu as pltpu
from jax.experimental.pallas import tpu_sc as plsc

N = 10000
E = 320000
E_ACT = E + N
NC = 2
NS = 16
KB = 128
NBLK = 81
T_EDGES = KB * NBLK
E_PAD = NC * NS * T_EDGES
ROW_CHUNK = 624
ROW_TAIL = N - NS * ROW_CHUNK
SHIFT = 8.0


def _edge_pass(xl, xr, src, dst, ea, att, wvec, zeros_blk, D, DR, W, ECOL):
    mesh = plsc.VectorSubcoreMesh(core_axis_name="c", subcore_axis_name="s")

    @functools.partial(
        pl.kernel,
        mesh=mesh,
        out_type=jax.ShapeDtypeStruct((NC, N, W), jnp.float32),
        compiler_params=pltpu.CompilerParams(needs_layout_passes=False,
                                             use_tc_tiling_on_sc=False),
        scratch_types=[
            pltpu.VMEM_SHARED((N, W), jnp.float32),
            pltpu.VMEM((KB,), jnp.int32),
            pltpu.VMEM((KB,), jnp.int32),
            pltpu.VMEM((KB,), jnp.float32),
            pltpu.VMEM((KB, DR), jnp.float32),
            pltpu.VMEM((KB, DR), jnp.float32),
            pltpu.VMEM((KB, W), jnp.float32),
            pltpu.VMEM((KB,), jnp.float32),
            pltpu.VMEM((DR,), jnp.float32),
            pltpu.VMEM((DR,), jnp.float32),
            pltpu.SemaphoreType.DMA,
        ],
    )
    def k(xl_hbm, xr_hbm, src_hbm, dst_hbm, ea_hbm, att_hbm, we_hbm, z_hbm,
          out_hbm, table, src_v, dst_v, ea_v, xlr, xrr, srows, ealpha_v,
          attv, wev, sem):
        c = lax.axis_index("c")
        s = lax.axis_index("s")
        wid = s * NC + c
        pltpu.sync_copy(att_hbm, attv)
        pltpu.sync_copy(we_hbm, wev)
        pltpu.sync_copy(z_hbm, table.at[pl.ds(s * ROW_CHUNK, ROW_CHUNK)])

        @pl.when(s == NS - 1)
        def _():
            pltpu.sync_copy(z_hbm.at[pl.ds(0, ROW_TAIL)],
                            table.at[pl.ds(NS * ROW_CHUNK, ROW_TAIL)])

        plsc.subcore_barrier()

        lanes = lax.iota(jnp.int32, 16)

        def block_body(blk, _):
            base = wid * T_EDGES + blk * KB
            pltpu.sync_copy(src_hbm.at[pl.ds(base, KB)], src_v)
            pltpu.sync_copy(dst_hbm.at[pl.ds(base, KB)], dst_v)
            pltpu.sync_copy(ea_hbm.at[pl.ds(base, KB)], ea_v)
            pltpu.async_copy(xl_hbm.at[src_v], xlr, sem).wait()
            pltpu.async_copy(xr_hbm.at[dst_v], xrr, sem).wait()

            def group_body(g, _):
                ridx = g * 16 + lanes
                eav = ea_v[pl.ds(g * 16, 16)]
                acc = jnp.zeros((16,), jnp.float32)
                for f in range(D):
                    fidx = jnp.full((16,), f, jnp.int32)
                    af = plsc.load_gather(attv, [fidx])
                    wf = plsc.load_gather(wev, [fidx])
                    xlv = plsc.load_gather(xlr, [ridx, fidx])
                    xrv = plsc.load_gather(xrr, [ridx, fidx])
                    m = xlv + xrv + eav * wf
                    m = jnp.maximum(m, 0.2 * m)
                    acc = acc + af * m
                gid = base + g * 16 + lanes
                ealpha = jnp.where(gid < E_ACT, jnp.exp(acc - SHIFT), 0.0)
                ealpha_v[pl.ds(g * 16, 16)] = ealpha
                return _

            lax.fori_loop(0, KB // 16, group_body, 0)

            def scale_body(e, _):
                sp = plsc.load_gather(ealpha_v, [jnp.full((16,), e, jnp.int32)])
                for j in range(W // 16):
                    b0 = j * 16
                    if b0 < DR:
                        v = xlr[e, pl.ds(b0, 16)] * sp
                    else:
                        v = jnp.zeros((16,), jnp.float32)
                    if b0 <= ECOL < b0 + 16:
                        oh = jnp.where(lanes == (ECOL - b0), 1.0, 0.0)
                        v = v + sp * oh
                    srows[e, pl.ds(b0, 16)] = v
                return _

            lax.fori_loop(0, KB, scale_body, 0)
            pltpu.sync_copy(srows, table.at[dst_v], add=True)
            return _

        lax.fori_loop(0, NBLK, block_body, 0)
        plsc.subcore_barrier()
        rows = pl.ds(s * ROW_CHUNK, ROW_CHUNK)
        pltpu.sync_copy(table.at[rows], out_hbm.at[c].at[rows])

        @pl.when(s == NS - 1)
        def _():
            tail = pl.ds(NS * ROW_CHUNK, ROW_TAIL)
            pltpu.sync_copy(table.at[tail], out_hbm.at[c].at[tail])

    return k(xl, xr, src, dst, ea, att, wvec, zeros_blk)


def _tc_pre(x, Wl, Wr, ea2d):
    def body(x_ref, wl_ref, wr_ref, ea_ref, xl_ref, xr_ref, mean_ref):
        xv = x_ref[...]
        xl_ref[...] = jnp.dot(xv, wl_ref[...], preferred_element_type=jnp.float32)
        xr_ref[...] = jnp.dot(xv, wr_ref[...], preferred_element_type=jnp.float32)
        mean_ref[...] = jnp.sum(ea_ref[...], keepdims=True).reshape(1, 1) / E

    return pl.pallas_call(
        body,
        out_shape=(
            jax.ShapeDtypeStruct((N, 64), jnp.float32),
            jax.ShapeDtypeStruct((N, 64), jnp.float32),
            jax.ShapeDtypeStruct((1, 1), jnp.float32),
        ),
    )(x, Wl, Wr, ea2d)


def _tc_mid(t, b1, Wl2p, Wr2p):
    def body(t_ref, b1_ref, wl_ref, wr_ref, hl_ref, hr_ref):
        acc = t_ref[0] + t_ref[1]
        num = acc[:, :64]
        den = acc[:, 64:65]
        h = jnp.maximum(num / (den + 1e-16) + b1_ref[...], 0.0)
        hl_ref[...] = jnp.dot(h, wl_ref[...], preferred_element_type=jnp.float32)
        hr_ref[...] = jnp.dot(h, wr_ref[...], preferred_element_type=jnp.float32)

    return pl.pallas_call(
        body,
        out_shape=(
            jax.ShapeDtypeStruct((N, 48), jnp.float32),
            jax.ShapeDtypeStruct((N, 48), jnp.float32),
        ),
    )(t, b1, Wl2p, Wr2p)


def _tc_post(t2, b2):
    def body(t_ref, b2_ref, o_ref):
        acc = t_ref[0] + t_ref[1]
        num = acc[:, :40]
        den = acc[:, 40:41]
        o = num / (den + 1e-16) + b2_ref[...]
        m = jnp.max(o, axis=1, keepdims=True)
        lse = m + jnp.log(jnp.sum(jnp.exp(o - m), axis=1, keepdims=True))
        o_ref[...] = o - lse

    return pl.pallas_call(
        body,
        out_shape=jax.ShapeDtypeStruct((N, 40), jnp.float32),
    )(t2, b2)


def kernel(x, G2_edge_attr, G1_edge_attr_matrix, G3_edge_index, G3_edge_attr,
           Wl1, Wr1, We1, att1, b1, Wl2, Wr2, We2, att2, b2):
    del G2_edge_attr, G1_edge_attr_matrix
    ea2d = G3_edge_attr.reshape(2500, 128)
    xl1, xr1, mean_ea = _tc_pre(x, Wl1, Wr1, ea2d)

    loop_idx = jnp.arange(N, dtype=jnp.int32)
    pad = E_PAD - E_ACT
    src = jnp.concatenate([G3_edge_index[0], loop_idx,
                           jnp.zeros((pad,), jnp.int32)])
    dst = jnp.concatenate([G3_edge_index[1], loop_idx,
                           jnp.zeros((pad,), jnp.int32)])
    ea = jnp.concatenate([G3_edge_attr[:, 0],
                          jnp.broadcast_to(mean_ea[0, 0], (N,)),
                          jnp.zeros((pad,), jnp.float32)])

    zeros80 = jnp.zeros((ROW_CHUNK, 80), jnp.float32)
    zeros48 = jnp.zeros((ROW_CHUNK, 48), jnp.float32)

    t1 = _edge_pass(xl1, xr1, src, dst, ea, att1, We1[0], zeros80,
                    D=64, DR=64, W=80, ECOL=64)

    pad8 = ((0, 0), (0, 8))
    hl2, hr2 = _tc_mid(t1, b1, jnp.pad(Wl2, pad8), jnp.pad(Wr2, pad8))

    att2p = jnp.pad(att2, (0, 8))
    we2p = jnp.pad(We2[0], (0, 8))
    t2 = _edge_pass(hl2, hr2, src, dst, ea, att2p, we2p, zeros48,
                    D=40, DR=48, W=48, ECOL=40)

    return _tc_post(t2, b2)


# speedup vs baseline: 9.4064x; 1.3854x over previous
import functools

import jax
import jax.numpy as jnp
from jax import lax
from jax.experimental import pallas as pl
from jax.experimental.pallas import tpu as pltpu
from jax.experimental.pallas import tpu_sc as plsc

N = 10000
E = 320000
E_ACT = E + N
NC = 2
NS = 16
KB = 128
NBLK = 84
T_EDGES = KB * NBLK
E_PAD = NC * NS * T_EDGES
ROW_CHUNK = 624
ROW_TAIL = N - NS * ROW_CHUNK
SHIFT = 8.0


def _edge_pass(xl, xr, src, dst, ea, att, wvec, zeros_blk, D, DR, W, ECOL):
    mesh = plsc.VectorSubcoreMesh(core_axis_name="c", subcore_axis_name="s")

    @functools.partial(
        pl.kernel,
        mesh=mesh,
        out_type=jax.ShapeDtypeStruct((NC, N, W), jnp.float32),
        compiler_params=pltpu.CompilerParams(needs_layout_passes=False,
                                             use_tc_tiling_on_sc=False),
        scratch_types=[
            pltpu.VMEM_SHARED((N, W), jnp.float32),
            [pltpu.VMEM((KB,), jnp.int32) for _ in range(4)],
            [pltpu.VMEM((KB,), jnp.int32) for _ in range(4)],
            [pltpu.VMEM((KB,), jnp.float32) for _ in range(4)],
            [pltpu.VMEM((KB, DR), jnp.float32) for _ in range(2)],
            [pltpu.VMEM((KB, DR), jnp.float32) for _ in range(2)],
            [pltpu.VMEM((KB, W), jnp.float32) for _ in range(2)],
            pltpu.VMEM((KB,), jnp.float32),
            pltpu.VMEM((DR,), jnp.float32),
            pltpu.VMEM((DR,), jnp.float32),
            [pltpu.SemaphoreType.DMA for _ in range(4)],
            [pltpu.SemaphoreType.DMA for _ in range(2)],
            [pltpu.SemaphoreType.DMA for _ in range(2)],
        ],
    )
    def k(xl_hbm, xr_hbm, src_hbm, dst_hbm, ea_hbm, att_hbm, we_hbm, z_hbm,
          out_hbm, table, src_v, dst_v, ea_v, xlr, xrr, srows, ealpha_v,
          attv, wev, sem_i, sem_g, sem_s):
        c = lax.axis_index("c")
        s = lax.axis_index("s")
        wid = s * NC + c
        pltpu.sync_copy(att_hbm, attv)
        pltpu.sync_copy(we_hbm, wev)
        pltpu.sync_copy(z_hbm, table.at[pl.ds(s * ROW_CHUNK, ROW_CHUNK)])

        @pl.when(s == NS - 1)
        def _():
            pltpu.sync_copy(z_hbm.at[pl.ds(0, ROW_TAIL)],
                            table.at[pl.ds(NS * ROW_CHUNK, ROW_TAIL)])

        plsc.subcore_barrier()

        lanes = lax.iota(jnp.int32, 16)
        tile_base = wid * T_EDGES

        def idx_base(blk):
            return tile_base + jnp.minimum(blk, NBLK - 1) * KB

        def issue_idx(blk, r):
            b0 = idx_base(blk)
            pltpu.async_copy(src_hbm.at[pl.ds(b0, KB)], src_v[r], sem_i[r])
            pltpu.async_copy(dst_hbm.at[pl.ds(b0, KB)], dst_v[r], sem_i[r])
            pltpu.async_copy(ea_hbm.at[pl.ds(b0, KB)], ea_v[r], sem_i[r])

        def wait_idx(blk, r):
            b0 = idx_base(blk)
            pltpu.make_async_copy(src_hbm.at[pl.ds(b0, KB)], src_v[r], sem_i[r]).wait()
            pltpu.make_async_copy(dst_hbm.at[pl.ds(b0, KB)], dst_v[r], sem_i[r]).wait()
            pltpu.make_async_copy(ea_hbm.at[pl.ds(b0, KB)], ea_v[r], sem_i[r]).wait()

        def issue_gather(r, p):
            pltpu.async_copy(xl_hbm.at[src_v[r]], xlr[p], sem_g[p])
            pltpu.async_copy(xr_hbm.at[dst_v[r]], xrr[p], sem_g[p])

        def wait_gather(r, p):
            pltpu.make_async_copy(xl_hbm.at[src_v[r]], xlr[p], sem_g[p]).wait()
            pltpu.make_async_copy(xr_hbm.at[dst_v[r]], xrr[p], sem_g[p]).wait()

        def compute(blk, r, p):
            base = tile_base + blk * KB

            def group_body(g, carry):
                ridx = g * 16 + lanes
                eav = ea_v[r][pl.ds(g * 16, 16)]
                acc = jnp.zeros((16,), jnp.float32)
                for f in range(D):
                    fidx = jnp.full((16,), f, jnp.int32)
                    af = plsc.load_gather(attv, [fidx])
                    wf = plsc.load_gather(wev, [fidx])
                    xlv = plsc.load_gather(xlr[p], [ridx, fidx])
                    xrv = plsc.load_gather(xrr[p], [ridx, fidx])
                    m = xlv + xrv + eav * wf
                    m = jnp.maximum(m, 0.2 * m)
                    acc = acc + af * m
                gid = base + g * 16 + lanes
                ealpha = jnp.where(gid < E_ACT, jnp.exp(acc - SHIFT), 0.0)
                ealpha_v[pl.ds(g * 16, 16)] = ealpha
                return carry

            lax.fori_loop(0, KB // 16, group_body, 0)

            def scale_body(e, carry):
                sp = plsc.load_gather(ealpha_v, [jnp.full((16,), e, jnp.int32)])
                for j in range(W // 16):
                    b0 = j * 16
                    if b0 < DR:
                        v = xlr[p][e, pl.ds(b0, 16)] * sp
                    else:
                        v = jnp.zeros((16,), jnp.float32)
                    if b0 <= ECOL < b0 + 16:
                        oh = jnp.where(lanes == (ECOL - b0), 1.0, 0.0)
                        v = v + sp * oh
                    srows[p][e, pl.ds(b0, 16)] = v
                return carry

            lax.fori_loop(0, KB, scale_body, 0)

        issue_idx(0, 0)
        issue_idx(1, 1)
        wait_idx(0, 0)
        issue_gather(0, 0)

        def quad_body(i, carry):
            bb = i * 4
            for j in range(4):
                blk = bb + j
                p = j % 2
                q = 1 - p
                r = j
                rn = (j + 1) % 4
                rp = (j + 2) % 4
                wait_gather(r, p)
                wait_idx(blk + 1, rn)
                issue_gather(rn, q)

                @pl.when(blk >= 2)
                def _():
                    pltpu.make_async_copy(
                        srows[p], table.at[dst_v[rp]], sem_s[p]).wait()

                issue_idx(blk + 2, rp)
                compute(blk, r, p)
                pltpu.async_copy(srows[p], table.at[dst_v[r]], sem_s[p],
                                 add=True)
            return carry

        lax.fori_loop(0, NBLK // 4, quad_body, 0)

        wait_gather(0, 0)
        wait_idx(NBLK + 1, 1)
        pltpu.make_async_copy(srows[0], table.at[dst_v[2]], sem_s[0]).wait()
        pltpu.make_async_copy(srows[1], table.at[dst_v[3]], sem_s[1]).wait()
        plsc.subcore_barrier()
        rows = pl.ds(s * ROW_CHUNK, ROW_CHUNK)
        pltpu.sync_copy(table.at[rows], out_hbm.at[c].at[rows])

        @pl.when(s == NS - 1)
        def _():
            tail = pl.ds(NS * ROW_CHUNK, ROW_TAIL)
            pltpu.sync_copy(table.at[tail], out_hbm.at[c].at[tail])

    return k(xl, xr, src, dst, ea, att, wvec, zeros_blk)


def _tc_pre(x, Wl, Wr, ea2d):
    def body(x_ref, wl_ref, wr_ref, ea_ref, xl_ref, xr_ref, mean_ref):
        xv = x_ref[...]
        xl_ref[...] = jnp.dot(xv, wl_ref[...], preferred_element_type=jnp.float32)
        xr_ref[...] = jnp.dot(xv, wr_ref[...], preferred_element_type=jnp.float32)
        mean_ref[...] = jnp.sum(ea_ref[...], keepdims=True).reshape(1, 1) / E

    return pl.pallas_call(
        body,
        out_shape=(
            jax.ShapeDtypeStruct((N, 64), jnp.float32),
            jax.ShapeDtypeStruct((N, 64), jnp.float32),
            jax.ShapeDtypeStruct((1, 1), jnp.float32),
        ),
    )(x, Wl, Wr, ea2d)


def _tc_mid(t, b1, Wl2p, Wr2p):
    def body(t_ref, b1_ref, wl_ref, wr_ref, hl_ref, hr_ref):
        acc = t_ref[0] + t_ref[1]
        num = acc[:, :64]
        den = acc[:, 64:65]
        h = jnp.maximum(num / (den + 1e-16) + b1_ref[...], 0.0)
        hl_ref[...] = jnp.dot(h, wl_ref[...], preferred_element_type=jnp.float32)
        hr_ref[...] = jnp.dot(h, wr_ref[...], preferred_element_type=jnp.float32)

    return pl.pallas_call(
        body,
        out_shape=(
            jax.ShapeDtypeStruct((N, 48), jnp.float32),
            jax.ShapeDtypeStruct((N, 48), jnp.float32),
        ),
    )(t, b1, Wl2p, Wr2p)


def _tc_post(t2, b2):
    def body(t_ref, b2_ref, o_ref):
        acc = t_ref[0] + t_ref[1]
        num = acc[:, :40]
        den = acc[:, 40:41]
        o = num / (den + 1e-16) + b2_ref[...]
        m = jnp.max(o, axis=1, keepdims=True)
        lse = m + jnp.log(jnp.sum(jnp.exp(o - m), axis=1, keepdims=True))
        o_ref[...] = o - lse

    return pl.pallas_call(
        body,
        out_shape=jax.ShapeDtypeStruct((N, 40), jnp.float32),
    )(t2, b2)


def kernel(x, G2_edge_attr, G1_edge_attr_matrix, G3_edge_index, G3_edge_attr,
           Wl1, Wr1, We1, att1, b1, Wl2, Wr2, We2, att2, b2):
    del G2_edge_attr, G1_edge_attr_matrix
    ea2d = G3_edge_attr.reshape(2500, 128)
    xl1, xr1, mean_ea = _tc_pre(x, Wl1, Wr1, ea2d)

    loop_idx = jnp.arange(N, dtype=jnp.int32)
    pad = E_PAD - E_ACT
    src = jnp.concatenate([G3_edge_index[0], loop_idx,
                           jnp.zeros((pad,), jnp.int32)])
    dst = jnp.concatenate([G3_edge_index[1], loop_idx,
                           jnp.zeros((pad,), jnp.int32)])
    ea = jnp.concatenate([G3_edge_attr[:, 0],
                          jnp.broadcast_to(mean_ea[0, 0], (N,)),
                          jnp.zeros((pad,), jnp.float32)])

    zeros80 = jnp.zeros((ROW_CHUNK, 80), jnp.float32)
    zeros48 = jnp.zeros((ROW_CHUNK, 48), jnp.float32)

    t1 = _edge_pass(xl1, xr1, src, dst, ea, att1, We1[0], zeros80,
                    D=64, DR=64, W=80, ECOL=64)

    pad8 = ((0, 0), (0, 8))
    hl2, hr2 = _tc_mid(t1, b1, jnp.pad(Wl2, pad8), jnp.pad(Wr2, pad8))

    att2p = jnp.pad(att2, (0, 8))
    we2p = jnp.pad(We2[0], (0, 8))
    t2 = _edge_pass(hl2, hr2, src, dst, ea, att2p, we2p, zeros48,
                    D=40, DR=48, W=48, ECOL=40)

    return _tc_post(t2, b2)


# speedup vs baseline: 11.5125x; 1.2239x over previous
import functools

import jax
import jax.numpy as jnp
from jax import lax
from jax.experimental import pallas as pl
from jax.experimental.pallas import tpu as pltpu
from jax.experimental.pallas import tpu_sc as plsc

N = 10000
E = 320000
E_ACT = E + N
NC = 2
NS = 16
KB = 128
NBLK = 84
T_EDGES = KB * NBLK
E_PAD = NC * NS * T_EDGES
ROW_CHUNK = 624
ROW_TAIL = N - NS * ROW_CHUNK
SHIFT = 8.0
W = 80


def _edge_pass(xl, xr, src, dst, ea, att, wvec, zeros_blk, D):
    mesh = plsc.VectorSubcoreMesh(core_axis_name="c", subcore_axis_name="s")

    @functools.partial(
        pl.kernel,
        mesh=mesh,
        out_type=jax.ShapeDtypeStruct((NC, N, W), jnp.float32),
        compiler_params=pltpu.CompilerParams(needs_layout_passes=False,
                                             use_tc_tiling_on_sc=False),
        scratch_types=[
            pltpu.VMEM_SHARED((N, W), jnp.float32),
            [pltpu.VMEM((KB,), jnp.int32) for _ in range(4)],
            [pltpu.VMEM((KB,), jnp.int32) for _ in range(4)],
            [pltpu.VMEM((KB,), jnp.float32) for _ in range(4)],
            [pltpu.VMEM((KB, 32), jnp.int32) for _ in range(2)],
            [pltpu.VMEM((KB, 32), jnp.int32) for _ in range(2)],
            [pltpu.VMEM((KB, W), jnp.float32) for _ in range(2)],
            pltpu.VMEM((KB,), jnp.float32),
            pltpu.VMEM((64,), jnp.float32),
            pltpu.VMEM((64,), jnp.float32),
            [pltpu.SemaphoreType.DMA for _ in range(4)],
            [pltpu.SemaphoreType.DMA for _ in range(2)],
            [pltpu.SemaphoreType.DMA for _ in range(2)],
        ],
    )
    def k(xl_hbm, xr_hbm, src_hbm, dst_hbm, ea_hbm, att_hbm, we_hbm, z_hbm,
          out_hbm, table, src_v, dst_v, ea_v, xlr, xrr, srows, ealpha_v,
          attv, wev, sem_i, sem_g, sem_s):
        c = lax.axis_index("c")
        s = lax.axis_index("s")
        wid = s * NC + c
        pltpu.sync_copy(att_hbm, attv)
        pltpu.sync_copy(we_hbm, wev)
        pltpu.sync_copy(z_hbm, table.at[pl.ds(s * ROW_CHUNK, ROW_CHUNK)])

        @pl.when(s == NS - 1)
        def _():
            pltpu.sync_copy(z_hbm.at[pl.ds(0, ROW_TAIL)],
                            table.at[pl.ds(NS * ROW_CHUNK, ROW_TAIL)])

        plsc.subcore_barrier()

        lanes = lax.iota(jnp.int32, 16)
        tile_base = wid * T_EDGES

        def idx_base(blk):
            return tile_base + jnp.minimum(blk, NBLK - 1) * KB

        def issue_idx(blk, r):
            b0 = idx_base(blk)
            pltpu.async_copy(src_hbm.at[pl.ds(b0, KB)], src_v[r], sem_i[r])
            pltpu.async_copy(dst_hbm.at[pl.ds(b0, KB)], dst_v[r], sem_i[r])
            pltpu.async_copy(ea_hbm.at[pl.ds(b0, KB)], ea_v[r], sem_i[r])

        def wait_idx(blk, r):
            b0 = idx_base(blk)
            pltpu.make_async_copy(src_hbm.at[pl.ds(b0, KB)], src_v[r], sem_i[r]).wait()
            pltpu.make_async_copy(dst_hbm.at[pl.ds(b0, KB)], dst_v[r], sem_i[r]).wait()
            pltpu.make_async_copy(ea_hbm.at[pl.ds(b0, KB)], ea_v[r], sem_i[r]).wait()

        def issue_gather(r, p):
            pltpu.async_copy(xl_hbm.at[src_v[r]], xlr[p], sem_g[p])
            pltpu.async_copy(xr_hbm.at[dst_v[r]], xrr[p], sem_g[p])

        def wait_gather(r, p):
            pltpu.make_async_copy(xl_hbm.at[src_v[r]], xlr[p], sem_g[p]).wait()
            pltpu.make_async_copy(xr_hbm.at[dst_v[r]], xrr[p], sem_g[p]).wait()

        himask = jnp.int32(-65536)

        def unpack2(cv):
            lo = plsc.bitcast(jnp.left_shift(cv, 16), jnp.float32)
            hi = plsc.bitcast(jnp.bitwise_and(cv, himask), jnp.float32)
            return lo, hi

        def compute(blk, r, p):
            base = tile_base + blk * KB
            rlist = [g * 16 + lanes for g in range(8)]
            eav_g = [ea_v[r][pl.ds(g * 16, 16)] for g in range(8)]

            def pair_body(fp, accs):
                f0 = 2 * fp
                a0 = plsc.load_gather(attv, [jnp.full((16,), f0, jnp.int32)])
                a1 = plsc.load_gather(attv, [jnp.full((16,), f0 + 1, jnp.int32)])
                w0 = plsc.load_gather(wev, [jnp.full((16,), f0, jnp.int32)])
                w1 = plsc.load_gather(wev, [jnp.full((16,), f0 + 1, jnp.int32)])
                fcol = jnp.full((16,), fp, jnp.int32)
                out = []
                for g in range(8):
                    xc = plsc.load_gather(xlr[p], [rlist[g], fcol])
                    rc = plsc.load_gather(xrr[p], [rlist[g], fcol])
                    xe, xo = unpack2(xc)
                    re_, ro = unpack2(rc)
                    m0 = xe + re_ + eav_g[g] * w0
                    m0 = jnp.maximum(m0, 0.2 * m0)
                    m1 = xo + ro + eav_g[g] * w1
                    m1 = jnp.maximum(m1, 0.2 * m1)
                    out.append(accs[g] + a0 * m0 + a1 * m1)
                return tuple(out)

            accs = lax.fori_loop(0, D // 2, pair_body,
                                 tuple(jnp.zeros((16,), jnp.float32)
                                       for _ in range(8)))
            for g in range(8):
                gid = base + g * 16 + lanes
                ealpha = jnp.where(gid < E_ACT, jnp.exp(accs[g] - SHIFT), 0.0)
                ealpha_v[pl.ds(g * 16, 16)] = ealpha

            oh0 = jnp.where(lanes == 0, 1.0, 0.0)

            def scale_body(e, carry):
                sp = plsc.load_gather(ealpha_v, [jnp.full((16,), e, jnp.int32)])
                for w in range(2):
                    cw = xlr[p][e, pl.ds(w * 16, 16)]
                    ve, vo = unpack2(cw)
                    srows[p][e, pl.ds(w * 16, 16)] = ve * sp
                    srows[p][e, pl.ds(32 + w * 16, 16)] = vo * sp
                srows[p][e, pl.ds(64, 16)] = sp * oh0
                return carry

            lax.fori_loop(0, KB, scale_body, 0)

        issue_idx(0, 0)
        issue_idx(1, 1)
        wait_idx(0, 0)
        issue_gather(0, 0)

        def quad_body(i, carry):
            bb = i * 4
            for j in range(4):
                blk = bb + j
                p = j % 2
                q = 1 - p
                r = j
                rn = (j + 1) % 4
                rp = (j + 2) % 4
                wait_gather(r, p)
                wait_idx(blk + 1, rn)
                issue_gather(rn, q)

                @pl.when(blk >= 2)
                def _():
                    pltpu.make_async_copy(
                        srows[p], table.at[dst_v[rp]], sem_s[p]).wait()

                issue_idx(blk + 2, rp)
                compute(blk, r, p)
                pltpu.async_copy(srows[p], table.at[dst_v[r]], sem_s[p],
                                 add=True)
            return carry

        lax.fori_loop(0, NBLK // 4, quad_body, 0)

        wait_gather(0, 0)
        wait_idx(NBLK + 1, 1)
        pltpu.make_async_copy(srows[0], table.at[dst_v[2]], sem_s[0]).wait()
        pltpu.make_async_copy(srows[1], table.at[dst_v[3]], sem_s[1]).wait()
        plsc.subcore_barrier()
        rows = pl.ds(s * ROW_CHUNK, ROW_CHUNK)
        pltpu.sync_copy(table.at[rows], out_hbm.at[c].at[rows])

        @pl.when(s == NS - 1)
        def _():
            tail = pl.ds(NS * ROW_CHUNK, ROW_TAIL)
            pltpu.sync_copy(table.at[tail], out_hbm.at[c].at[tail])

    return k(xl, xr, src, dst, ea, att, wvec, zeros_blk)


def _tc_pre(x, Wl, Wr, ea2d):
    def body(x_ref, wl_ref, wr_ref, ea_ref, xl_ref, xr_ref, mean_ref):
        xv = x_ref[...]
        xl_ref[...] = jnp.dot(xv, wl_ref[...], preferred_element_type=jnp.float32)
        xr_ref[...] = jnp.dot(xv, wr_ref[...], preferred_element_type=jnp.float32)
        mean_ref[...] = jnp.sum(ea_ref[...], keepdims=True).reshape(1, 1) / E

    return pl.pallas_call(
        body,
        out_shape=(
            jax.ShapeDtypeStruct((N, 64), jnp.float32),
            jax.ShapeDtypeStruct((N, 64), jnp.float32),
            jax.ShapeDtypeStruct((1, 1), jnp.float32),
        ),
    )(x, Wl, Wr, ea2d)


def _tc_mid(t, b1p, Wl2p, Wr2p):
    def body(t_ref, b1_ref, wl_ref, wr_ref, hl_ref, hr_ref):
        acc = t_ref[0] + t_ref[1]
        num = acc[:, :64]
        den = acc[:, 64:65]
        h = jnp.maximum(num / (den + 1e-16) + b1_ref[...], 0.0)
        hl_ref[...] = jnp.dot(h, wl_ref[...], preferred_element_type=jnp.float32)
        hr_ref[...] = jnp.dot(h, wr_ref[...], preferred_element_type=jnp.float32)

    return pl.pallas_call(
        body,
        out_shape=(
            jax.ShapeDtypeStruct((N, 64), jnp.float32),
            jax.ShapeDtypeStruct((N, 64), jnp.float32),
        ),
    )(t, b1p, Wl2p, Wr2p)


def _tc_post(t2, b2p, maskp):
    def body(t_ref, b2_ref, mask_ref, o_ref):
        acc = t_ref[0] + t_ref[1]
        num = acc[:, :64]
        den = acc[:, 64:65]
        o = num / (den + 1e-16) + b2_ref[...]
        om = jnp.where(mask_ref[...] > 0, o, -1e30)
        m = jnp.max(om, axis=1, keepdims=True)
        lse = m + jnp.log(jnp.sum(jnp.exp(om - m), axis=1, keepdims=True))
        o_ref[...] = o - lse

    return pl.pallas_call(
        body,
        out_shape=jax.ShapeDtypeStruct((N, 64), jnp.float32),
    )(t2, b2p, maskp)


def kernel(x, G2_edge_attr, G1_edge_attr_matrix, G3_edge_index, G3_edge_attr,
           Wl1, Wr1, We1, att1, b1, Wl2, Wr2, We2, att2, b2):
    del G2_edge_attr, G1_edge_attr_matrix
    ea2d = G3_edge_attr.reshape(2500, 128)
    xl1, xr1, mean_ea = _tc_pre(x, Wl1, Wr1, ea2d)

    loop_idx = jnp.arange(N, dtype=jnp.int32)
    pad = E_PAD - E_ACT
    src = jnp.concatenate([G3_edge_index[0], loop_idx,
                           jnp.zeros((pad,), jnp.int32)])
    dst = jnp.concatenate([G3_edge_index[1], loop_idx,
                           jnp.zeros((pad,), jnp.int32)])
    ea = jnp.concatenate([G3_edge_attr[:, 0],
                          jnp.broadcast_to(mean_ea[0, 0], (N,)),
                          jnp.zeros((pad,), jnp.float32)])

    zeros80 = jnp.zeros((ROW_CHUNK, W), jnp.float32)

    def pack(a):
        return lax.bitcast_convert_type(
            a.astype(jnp.bfloat16).reshape(N, 32, 2), jnp.int32)

    t1 = _edge_pass(pack(xl1), pack(xr1), src, dst, ea, att1, We1[0],
                    zeros80, D=64)

    perm = jnp.concatenate([jnp.arange(0, 64, 2), jnp.arange(1, 64, 2)])
    pad24 = ((0, 0), (0, 24))
    hl2, hr2 = _tc_mid(t1, b1[perm], jnp.pad(Wl2[perm, :], pad24),
                       jnp.pad(Wr2[perm, :], pad24))

    att2p = jnp.pad(att2, (0, 24))
    we2p = jnp.pad(We2[0], (0, 24))
    t2 = _edge_pass(pack(hl2), pack(hr2), src, dst, ea, att2p, we2p,
                    zeros80, D=40)

    b2p = jnp.pad(b2, (0, 24))[perm]
    maskp = (jnp.pad(jnp.ones((1, 40), jnp.float32), pad24))[:, perm]
    outp = _tc_post(t2, b2p, maskp)

    full = jnp.stack([outp[:, :32], outp[:, 32:64]], axis=2).reshape(N, 64)
    return full[:, :40]
